# Initial kernel scaffold; baseline (speedup 1.0000x reference)
import jax
import jax.numpy as jnp
from jax.experimental import pallas as pl


def kernel(x, edge_index, W1, b1, W2, b2):
    raise NotImplementedError("write your pallas kernel here")


# speedup vs baseline: 15.6154x; 15.6154x over previous
import functools

import jax
import jax.numpy as jnp
from jax import lax
from jax.experimental import pallas as pl
from jax.experimental.pallas import tpu as pltpu
from jax.experimental.pallas import tpu_sc as plsc

_NC = 2
_NS = 16
_CH = 128
_NP = 10240
_ZR = 128
_TCB = 1000


def _make_degree(n, e):
  nc, ns, ch = _NC, _NS, _CH
  ew = e // (nc * ns)
  nfull = ew // ch
  rem = ew - nfull * ch
  npad = _NP
  rpt = npad // ns
  zr = _ZR
  nz = rpt // zr
  w = 16
  mesh = plsc.VectorSubcoreMesh(core_axis_name="c", subcore_axis_name="s")

  @functools.partial(
      pl.kernel,
      mesh=mesh,
      out_type=jax.ShapeDtypeStruct((nc * npad, w), jnp.float32),
      scratch_types=[
          pltpu.VMEM((ch,), jnp.int32),
          pltpu.VMEM((max(rem, 8),), jnp.int32),
          pltpu.VMEM((ch, w), jnp.float32),
          pltpu.VMEM((zr, w), jnp.float32),
          pltpu.VMEM_SHARED((npad, w), jnp.float32),
      ],
  )
  def deg_kernel(dst_hbm, out_hbm, didx, didx_r, ones_v, zbuf, acc):
    c = lax.axis_index("c")
    s = lax.axis_index("s")

    def fill_ones(i, _):
      ones_v[i, :] = jnp.ones((16,), jnp.float32)
      return 0

    lax.fori_loop(0, ch, fill_ones, 0)

    def fill_zeros(i, _):
      zbuf[i, :] = jnp.zeros((16,), jnp.float32)
      return 0

    lax.fori_loop(0, zr, fill_zeros, 0)

    def zcp(k, _):
      pltpu.sync_copy(zbuf, acc.at[pl.ds(s * rpt + k * zr, zr)])
      return 0

    lax.fori_loop(0, nz, zcp, 0)
    plsc.subcore_barrier()

    base = (c * ns + s) * ew

    def chunk(k, _):
      pltpu.sync_copy(dst_hbm.at[pl.ds(base + k * ch, ch)], didx)
      pltpu.sync_copy(ones_v, acc.at[didx], add=True)
      return 0

    lax.fori_loop(0, nfull, chunk, 0)
    if rem:
      pltpu.sync_copy(dst_hbm.at[pl.ds(base + nfull * ch, rem)], didx_r)
      pltpu.sync_copy(ones_v.at[pl.ds(0, rem)], acc.at[didx_r], add=True)
    plsc.subcore_barrier()

    def wout(k, _):
      r = s * rpt + k * zr
      pltpu.sync_copy(acc.at[pl.ds(r, zr)],
                      out_hbm.at[pl.ds(c * npad + r, zr)])
      return 0

    lax.fori_loop(0, nz, wout, 0)

  return deg_kernel


def _make_agg(n, e, d):
  nc, ns, ch = _NC, _NS, _CH
  ew = e // (nc * ns)
  nfull = ew // ch
  rem = ew - nfull * ch
  npad = _NP
  rpt = npad // ns
  zr = _ZR
  nz = rpt // zr
  mesh = plsc.VectorSubcoreMesh(core_axis_name="c", subcore_axis_name="s")

  @functools.partial(
      pl.kernel,
      mesh=mesh,
      out_type=jax.ShapeDtypeStruct((nc * npad, d), jnp.float32),
      scratch_types=[
          pltpu.VMEM((ch,), jnp.int32),
          pltpu.VMEM((ch,), jnp.int32),
          pltpu.VMEM((max(rem, 8),), jnp.int32),
          pltpu.VMEM((max(rem, 8),), jnp.int32),
          pltpu.VMEM((ch, d), jnp.float32),
          pltpu.VMEM((zr, d), jnp.float32),
          pltpu.VMEM_SHARED((npad, d), jnp.float32),
          pltpu.SemaphoreType.DMA,
      ],
  )
  def agg_kernel(h_hbm, src_hbm, dst_hbm, out_hbm, sidx, didx, sidx_r,
                 didx_r, rows, zbuf, acc, sem):
    c = lax.axis_index("c")
    s = lax.axis_index("s")

    def fz(i, _):
      def fz2(j, _):
        zbuf[i, pl.ds(j * 16, 16)] = jnp.zeros((16,), jnp.float32)
        return 0

      return lax.fori_loop(0, d // 16, fz2, 0)

    lax.fori_loop(0, zr, fz, 0)

    def zcp(k, _):
      pltpu.sync_copy(zbuf, acc.at[pl.ds(s * rpt + k * zr, zr)])
      return 0

    lax.fori_loop(0, nz, zcp, 0)
    plsc.subcore_barrier()

    base = (c * ns + s) * ew

    def chunk(k, _):
      off = base + k * ch
      pltpu.sync_copy(src_hbm.at[pl.ds(off, ch)], sidx)
      pltpu.sync_copy(dst_hbm.at[pl.ds(off, ch)], didx)
      pltpu.async_copy(h_hbm.at[sidx], rows, sem).wait()
      pltpu.sync_copy(rows, acc.at[didx], add=True)
      return 0

    lax.fori_loop(0, nfull, chunk, 0)
    if rem:
      off = base + nfull * ch
      pltpu.sync_copy(src_hbm.at[pl.ds(off, rem)], sidx_r)
      pltpu.sync_copy(dst_hbm.at[pl.ds(off, rem)], didx_r)
      pltpu.async_copy(h_hbm.at[sidx_r], rows.at[pl.ds(0, rem)], sem).wait()
      pltpu.sync_copy(rows.at[pl.ds(0, rem)], acc.at[didx_r], add=True)
    plsc.subcore_barrier()

    def wout(k, _):
      r = s * rpt + k * zr
      pltpu.sync_copy(acc.at[pl.ds(r, zr)],
                      out_hbm.at[pl.ds(c * npad + r, zr)])
      return 0

    lax.fori_loop(0, nz, wout, 0)

  return agg_kernel


def _dinv_of(deg_ref):
  dsum = (deg_ref[0] + deg_ref[1])[:, 0:1] + 1.0
  return lax.rsqrt(dsum)


def _tc1_body(x_ref, w_ref, deg_ref, o_ref):
  dinv = _dinv_of(deg_ref)
  h = jnp.dot(x_ref[...], w_ref[...], preferred_element_type=jnp.float32)
  o_ref[...] = h * dinv


def _tc2_body(a_ref, h_ref, deg_ref, w_ref, b_ref, o_ref):
  dinv = _dinv_of(deg_ref)
  tot = a_ref[0] + a_ref[1] + h_ref[...]
  g = jnp.maximum(tot * dinv + b_ref[...], 0.0)
  h2 = jnp.dot(g, w_ref[...], preferred_element_type=jnp.float32)
  o_ref[...] = h2 * dinv


def _tc3_body(a_ref, h_ref, deg_ref, b_ref, o_ref):
  dinv = _dinv_of(deg_ref)
  d_out = o_ref.shape[1]
  t = ((a_ref[0] + a_ref[1] + h_ref[...]) * dinv)[:, :d_out] + b_ref[...]
  m = jnp.max(t, axis=1, keepdims=True)
  ex = t - m
  lse = jnp.log(jnp.sum(jnp.exp(ex), axis=1, keepdims=True))
  o_ref[...] = ex - lse


def _tc1(x, w1, deg3):
  n, d_in = x.shape
  d_hid = w1.shape[1]
  b = _TCB
  return pl.pallas_call(
      _tc1_body,
      grid=(n // b,),
      in_specs=[
          pl.BlockSpec((b, d_in), lambda i: (i, 0)),
          pl.BlockSpec((d_in, d_hid), lambda i: (0, 0)),
          pl.BlockSpec((2, b, 16), lambda i: (0, i, 0)),
      ],
      out_specs=pl.BlockSpec((b, d_hid), lambda i: (i, 0)),
      out_shape=jax.ShapeDtypeStruct((n, d_hid), jnp.float32),
  )(x, w1, deg3)


def _tc2(acc3, h1p, deg3, w2, b1):
  n, d_hid = h1p.shape
  d_out = w2.shape[1]
  b = _TCB
  return pl.pallas_call(
      _tc2_body,
      grid=(n // b,),
      in_specs=[
          pl.BlockSpec((2, b, d_hid), lambda i: (0, i, 0)),
          pl.BlockSpec((b, d_hid), lambda i: (i, 0)),
          pl.BlockSpec((2, b, 16), lambda i: (0, i, 0)),
          pl.BlockSpec((d_hid, d_out), lambda i: (0, 0)),
          pl.BlockSpec((1, d_hid), lambda i: (0, 0)),
      ],
      out_specs=pl.BlockSpec((b, d_out), lambda i: (i, 0)),
      out_shape=jax.ShapeDtypeStruct((n, d_out), jnp.float32),
  )(acc3, h1p, deg3, w2, b1)


def _tc3(acc3, h2p, deg3, b2):
  n, d_pad = h2p.shape
  d_out = b2.shape[1]
  b = _TCB
  return pl.pallas_call(
      _tc3_body,
      grid=(n // b,),
      in_specs=[
          pl.BlockSpec((2, b, d_pad), lambda i: (0, i, 0)),
          pl.BlockSpec((b, d_pad), lambda i: (i, 0)),
          pl.BlockSpec((2, b, 16), lambda i: (0, i, 0)),
          pl.BlockSpec((1, d_out), lambda i: (0, 0)),
      ],
      out_specs=pl.BlockSpec((b, d_out), lambda i: (i, 0)),
      out_shape=jax.ShapeDtypeStruct((n, d_out), jnp.float32),
  )(acc3, h2p, deg3, b2)


def kernel(x, edge_index, W1, b1, W2, b2):
  n, _ = x.shape
  e = edge_index.shape[1]
  d_hid = W1.shape[1]
  d_out = W2.shape[1]
  ei = edge_index.astype(jnp.int32)
  src = ei[0]
  dst = ei[1]

  deg = _make_degree(n, e)(dst)
  deg3 = deg.reshape(2, _NP, 16)[:, :n]

  d_pad = 128
  w2p = jnp.concatenate(
      [W2, jnp.zeros((d_hid, d_pad - d_out), jnp.float32)], axis=1)

  h1p = _tc1(x, W1, deg3)
  acc1 = _make_agg(n, e, d_hid)(h1p, src, dst)
  h2p = _tc2(acc1.reshape(2, _NP, d_hid)[:, :n], h1p, deg3, w2p,
             b1.reshape(1, d_hid))
  acc2 = _make_agg(n, e, d_pad)(h2p, src, dst)
  return _tc3(acc2.reshape(2, _NP, d_pad)[:, :n], h2p, deg3,
              b2.reshape(1, d_out))
